# baseline (device time: 25245 ns/iter reference)
import jax
import jax.numpy as jnp
from jax import lax
from jax.experimental import pallas as pl
from jax.experimental.pallas import tpu as pltpu

N_GLOBAL = 2048
EPS = 1e-5


def kernel(x, gamma, beta):
    m, n_loc = x.shape

    def body(x_ref, gamma_ref, beta_ref, out_ref, stats_ref, send_sem, recv_sem):
        my_x = lax.axis_index("x")
        my_y = lax.axis_index("y")
        peer = (my_x, 1 - my_y)

        barrier_sem = pltpu.get_barrier_semaphore()
        pl.semaphore_signal(
            barrier_sem, inc=1, device_id=peer,
            device_id_type=pl.DeviceIdType.MESH,
        )
        pl.semaphore_wait(barrier_sem, 1)

        xv = x_ref[:, :]
        s1 = jnp.sum(xv, axis=1)
        s2 = jnp.sum(xv * xv, axis=1)
        stats_ref[0, 0:1, :] = s1.reshape(1, m)
        stats_ref[0, 1:2, :] = s2.reshape(1, m)

        rdma = pltpu.make_async_remote_copy(
            src_ref=stats_ref.at[0],
            dst_ref=stats_ref.at[1],
            send_sem=send_sem,
            recv_sem=recv_sem,
            device_id=peer,
            device_id_type=pl.DeviceIdType.MESH,
        )
        rdma.start()
        rdma.wait()

        tot1 = stats_ref[0, 0:1, :] + stats_ref[1, 0:1, :]
        tot2 = stats_ref[0, 1:2, :] + stats_ref[1, 1:2, :]
        mean_r = tot1 / N_GLOBAL
        var_r = tot2 / N_GLOBAL - mean_r * mean_r
        rstd_r = lax.rsqrt(var_r + EPS)
        mean_c = mean_r.reshape(m, 1)
        rstd_c = rstd_r.reshape(m, 1)
        out_ref[:, :] = (
            (x_ref[:, :] - mean_c) * rstd_c * gamma_ref[0:1, :]
            + beta_ref[0:1, :]
        )

    return pl.pallas_call(
        body,
        out_shape=jax.ShapeDtypeStruct((m, n_loc), jnp.float32),
        in_specs=[
            pl.BlockSpec(memory_space=pltpu.VMEM),
            pl.BlockSpec(memory_space=pltpu.VMEM),
            pl.BlockSpec(memory_space=pltpu.VMEM),
        ],
        out_specs=pl.BlockSpec(memory_space=pltpu.VMEM),
        scratch_shapes=[
            pltpu.VMEM((2, 2, m), jnp.float32),
            pltpu.SemaphoreType.DMA,
            pltpu.SemaphoreType.DMA,
        ],
        compiler_params=pltpu.CompilerParams(collective_id=0),
    )(x, gamma.reshape(1, n_loc), beta.reshape(1, n_loc))
